# baseline (speedup 1.0000x reference)
import functools

import jax
import jax.numpy as jnp
from jax import lax
from jax.experimental import pallas as pl
from jax.experimental.pallas import tpu as pltpu
from jax.experimental.pallas import tpu_sc as plsc

N = 10000
NP = 10240
E = 160000
EP = 163840
D = 256
H = 256
C = 64
NC, NS = 2, 16
CH = 128
CHUNKS = EP // (NS * CH)
RPT = NP // NS
BN = 256
W1 = H // 4
W2 = C // 2


def _tc1_body(x_ref, wl_ref, wr_ref, b1_ref, *out_refs):
    t_refs, r1_ref = out_refs[:4], out_refs[4]
    xb = x_ref[...]
    p1 = jnp.dot(xb, wl_ref[...], preferred_element_type=jnp.float32)
    for q in range(4):
        t_refs[q][...] = p1[:, q * W1:(q + 1) * W1]
    r1_ref[...] = jnp.dot(xb, wr_ref[...], preferred_element_type=jnp.float32) + b1_ref[...]


def _tc2_body(a0_ref, a1_ref, a2_ref, a3_ref, ca_ref, cb_ref, r1_ref,
              wl_ref, wr_ref, b2_ref, *out_refs):
    t_refs, r2_ref = out_refs[:2], out_refs[2]
    cnt = ca_ref[...][:, 0:1] + cb_ref[...][:, 0:1]
    inv = 1.0 / jnp.maximum(cnt, 1.0)
    agg = jnp.concatenate(
        [a0_ref[...], a1_ref[...], a2_ref[...], a3_ref[...]], axis=1)
    h = jnp.maximum(agg * inv + r1_ref[...], 0.0)
    p2 = jnp.dot(h, wl_ref[...], preferred_element_type=jnp.float32)
    for q in range(2):
        t_refs[q][...] = p2[:, q * W2:(q + 1) * W2]
    r2_ref[...] = jnp.dot(h, wr_ref[...], preferred_element_type=jnp.float32) + b2_ref[...]


def _tc3_body(o0_ref, o1_ref, ca_ref, cb_ref, r2_ref, out_ref):
    cnt = ca_ref[...][:, 0:1] + cb_ref[...][:, 0:1]
    inv = 1.0 / jnp.maximum(cnt, 1.0)
    agg = jnp.concatenate([o0_ref[...], o1_ref[...]], axis=1)
    out_ref[...] = agg * inv + r2_ref[...]


def _tc1(x_pad, w1l_t, w1r_t, b1_row):
    blk = lambda i: (i, 0)
    full = lambda i: (0, 0)
    return pl.pallas_call(
        _tc1_body,
        grid=(NP // BN,),
        in_specs=[
            pl.BlockSpec((BN, D), blk),
            pl.BlockSpec((D, H), full),
            pl.BlockSpec((D, H), full),
            pl.BlockSpec((1, H), full),
        ],
        out_specs=[pl.BlockSpec((BN, W1), blk)] * 4 + [pl.BlockSpec((BN, H), blk)],
        out_shape=[jax.ShapeDtypeStruct((NP, W1), jnp.float32)] * 4
        + [jax.ShapeDtypeStruct((NP, H), jnp.float32)],
    )(x_pad, w1l_t, w1r_t, b1_row)


def _tc2(aggs, ca, cb, r1, w2l_t, w2r_t, b2_row):
    blk = lambda i: (i, 0)
    full = lambda i: (0, 0)
    return pl.pallas_call(
        _tc2_body,
        grid=(NP // BN,),
        in_specs=[pl.BlockSpec((BN, W1), blk)] * 4
        + [pl.BlockSpec((BN, 16), blk)] * 2
        + [
            pl.BlockSpec((BN, H), blk),
            pl.BlockSpec((H, C), full),
            pl.BlockSpec((H, C), full),
            pl.BlockSpec((1, C), full),
        ],
        out_specs=[pl.BlockSpec((BN, W2), blk)] * 2 + [pl.BlockSpec((BN, C), blk)],
        out_shape=[jax.ShapeDtypeStruct((NP, W2), jnp.float32)] * 2
        + [jax.ShapeDtypeStruct((NP, C), jnp.float32)],
    )(*aggs, ca, cb, r1, w2l_t, w2r_t, b2_row)


def _tc3(os, ca, cb, r2):
    bn3 = 400
    blk = lambda i: (i, 0)
    return pl.pallas_call(
        _tc3_body,
        grid=(N // bn3,),
        in_specs=[pl.BlockSpec((bn3, W2), blk)] * 2
        + [pl.BlockSpec((bn3, 16), blk)] * 2
        + [pl.BlockSpec((bn3, C), blk)],
        out_specs=pl.BlockSpec((bn3, C), blk),
        out_shape=jax.ShapeDtypeStruct((N, C), jnp.float32),
    )(*os, ca, cb, r2)


@functools.cache
def _make_sc_agg(w, groups, with_counts):
    passes = groups // NC
    mesh = plsc.VectorSubcoreMesh(
        core_axis_name="c", subcore_axis_name="s", num_cores=NC, num_subcores=NS)

    out_type = [jax.ShapeDtypeStruct((NP, w), jnp.float32) for _ in range(groups)]
    scratch = [
        pltpu.VMEM((CHUNKS, CH), jnp.int32),
        pltpu.VMEM((CHUNKS, CH), jnp.int32),
        pltpu.VMEM((CH, w), jnp.float32),
        pltpu.VMEM((CH, w), jnp.float32),
        pltpu.VMEM_SHARED((NP, w), jnp.float32),
        pltpu.SemaphoreType.DMA,
        pltpu.SemaphoreType.DMA,
        pltpu.SemaphoreType.DMA,
        pltpu.SemaphoreType.DMA,
    ]
    if with_counts:
        out_type += [jax.ShapeDtypeStruct((NP, 16), jnp.float32)] * 2
        scratch += [
            pltpu.VMEM((CH, 16), jnp.float32),
            pltpu.VMEM_SHARED((NP, 16), jnp.float32),
        ]

    @functools.partial(
        pl.kernel, out_type=tuple(out_type), mesh=mesh,
        scratch_types=tuple(scratch),
        compiler_params=pltpu.CompilerParams(use_tc_tiling_on_sc=False))
    def sc_agg(*args):
        tbls = args[:groups]
        srcix, dstix, zrow = args[groups:groups + 3]
        rest = args[groups + 3:]
        if with_counts:
            zcnt, ones_hbm = rest[0:2]
            rest = rest[2:]
        outs = rest[:groups]
        rest = rest[groups:]
        if with_counts:
            cnt_a, cnt_b = rest[0:2]
            src_v, dst_v, r0, r1, acc, s0, s1, t0, t1, ones_v, cacc = rest[2:]
        else:
            src_v, dst_v, r0, r1, acc, s0, s1, t0, t1 = rest
        c = lax.axis_index("c")
        s = lax.axis_index("s")
        rows = pl.ds(s * RPT, RPT)
        pltpu.sync_copy(srcix.at[pl.ds(s * CHUNKS, CHUNKS)], src_v)
        pltpu.sync_copy(dstix.at[pl.ds(s * CHUNKS, CHUNKS)], dst_v)
        if with_counts:
            pltpu.sync_copy(ones_hbm, ones_v)
            pltpu.sync_copy(zcnt, cacc.at[rows])

        def run_pass(tbl, do_counts):
            pltpu.async_copy(tbl.at[src_v.at[0]], r0, s0)

            def body(i, carry):
                j0 = 2 * i
                pltpu.make_async_copy(tbl.at[src_v.at[j0]], r0, s0).wait()

                @pl.when(i > 0)
                def _():
                    pltpu.make_async_copy(
                        r1, acc.at[dst_v.at[j0 - 1]], t1).wait()

                pltpu.async_copy(tbl.at[src_v.at[j0 + 1]], r1, s1)
                pltpu.async_copy(r0, acc.at[dst_v.at[j0]], t0, add=True)
                if do_counts == 1:
                    pltpu.sync_copy(ones_v, cacc.at[dst_v.at[j0]], add=True)
                pltpu.make_async_copy(tbl.at[src_v.at[j0 + 1]], r1, s1).wait()
                pltpu.make_async_copy(r0, acc.at[dst_v.at[j0]], t0).wait()

                @pl.when(i < CHUNKS // 2 - 1)
                def _():
                    pltpu.async_copy(tbl.at[src_v.at[j0 + 2]], r0, s0)

                pltpu.async_copy(r1, acc.at[dst_v.at[j0 + 1]], t1, add=True)
                if do_counts == 2:
                    pltpu.sync_copy(ones_v, cacc.at[dst_v.at[j0 + 1]], add=True)
                return carry

            lax.fori_loop(0, CHUNKS // 2, body, 0)
            pltpu.make_async_copy(r1, acc.at[dst_v.at[CHUNKS - 1]], t1).wait()

        for p in range(passes):
            pltpu.sync_copy(zrow, acc.at[rows])
            plsc.subcore_barrier()
            dc = with_counts and p == 0

            @pl.when(c == 0)
            def _():
                run_pass(tbls[p], 1 if dc else 0)

            @pl.when(c == 1)
            def _():
                run_pass(tbls[passes + p], 2 if dc else 0)

            plsc.subcore_barrier()

            out_c0 = outs[p]
            out_c1 = outs[passes + p]

            @pl.when(c == 0)
            def _():
                pltpu.sync_copy(acc.at[rows], out_c0.at[rows])

            @pl.when(c == 1)
            def _():
                pltpu.sync_copy(acc.at[rows], out_c1.at[rows])

        if with_counts:
            @pl.when(c == 0)
            def _():
                pltpu.sync_copy(cacc.at[rows], cnt_a.at[rows])

            @pl.when(c == 1)
            def _():
                pltpu.sync_copy(cacc.at[rows], cnt_b.at[rows])

    return sc_agg


def _sc_agg1(tbls, srcix, dstix, zrow, zcnt, ones_hbm):
    return _make_sc_agg(W1, 4, True)(*tbls, srcix, dstix, zrow, zcnt, ones_hbm)


def _sc_agg2(tbls, srcix, dstix, zrow):
    return _make_sc_agg(W2, 2, False)(*tbls, srcix, dstix, zrow)


def kernel(x, edge_index, W1_l, b1_l, W1_r, W2_l, b2_l, W2_r):
    x = x.astype(jnp.float32)
    src = edge_index[0].astype(jnp.int32)
    dst = edge_index[1].astype(jnp.int32)

    npad = EP - E
    pad_src = (lax.iota(jnp.int32, npad) * 37) % N
    pad_dst = N + lax.rem(lax.iota(jnp.int32, npad), NP - N)
    src_p = jnp.concatenate([src, pad_src])
    dst_p = jnp.concatenate([dst, pad_dst])

    src_ix = src_p.reshape(NS * CHUNKS, CH)
    dst_ix = dst_p.reshape(NS * CHUNKS, CH)

    x_pad = jnp.concatenate([x, jnp.zeros((NP - N, D), jnp.float32)])
    w1l_t = W1_l.T
    w1r_t = W1_r.T
    w2l_t = W2_l.T
    w2r_t = W2_r.T
    b1_row = b1_l.reshape(1, H)
    b2_row = b2_l.reshape(1, C)

    ones_rows = jnp.concatenate(
        [jnp.ones((CH, 1), jnp.float32), jnp.zeros((CH, 15), jnp.float32)], axis=1)
    zrow1 = jnp.zeros((RPT, W1), jnp.float32)
    zcnt = jnp.zeros((RPT, 16), jnp.float32)
    zrow2 = jnp.zeros((RPT, W2), jnp.float32)

    t1 = _tc1(x_pad, w1l_t, w1r_t, b1_row)
    r1 = t1[4]

    a0, a1, a2, a3, cnt_a, cnt_b = _sc_agg1(
        t1[:4], src_ix, dst_ix, zrow1, zcnt, ones_rows)

    t2 = _tc2((a0, a1, a2, a3), cnt_a, cnt_b, r1, w2l_t, w2r_t, b2_row)
    r2 = t2[2]

    o0, o1 = _sc_agg2(t2[:2], src_ix, dst_ix, zrow2)

    return _tc3((o0, o1), cnt_a, cnt_b, r2)


# speedup vs baseline: 1.0006x; 1.0006x over previous
import functools

import jax
import jax.numpy as jnp
from jax import lax
from jax.experimental import pallas as pl
from jax.experimental.pallas import tpu as pltpu
from jax.experimental.pallas import tpu_sc as plsc

N = 10000
NP = 10240
E = 160000
EP = 163840
D = 256
H = 256
C = 64
NC, NS = 2, 16
CH = 128
CHUNKS = EP // (NS * CH)
RPT = NP // NS
BN = 256
W1 = H // 4
W2 = C // 2


def _tc1_body(x_ref, wl_ref, wr_ref, b1_ref, *out_refs):
    t_refs, r1_ref = out_refs[:4], out_refs[4]
    xb = x_ref[...]
    p1 = jnp.dot(xb, wl_ref[...], preferred_element_type=jnp.float32)
    for q in range(4):
        t_refs[q][...] = p1[:, q * W1:(q + 1) * W1]
    r1_ref[...] = jnp.dot(xb, wr_ref[...], preferred_element_type=jnp.float32) + b1_ref[...]


def _tc2_body(a0_ref, a1_ref, a2_ref, a3_ref, ca_ref, cb_ref, r1_ref,
              wl_ref, wr_ref, b2_ref, *out_refs):
    t_refs, r2_ref = out_refs[:2], out_refs[2]
    cnt = ca_ref[...][:, 0:1] + cb_ref[...][:, 0:1]
    inv = 1.0 / jnp.maximum(cnt, 1.0)
    agg = jnp.concatenate(
        [a0_ref[...], a1_ref[...], a2_ref[...], a3_ref[...]], axis=1)
    h = jnp.maximum(agg * inv + r1_ref[...], 0.0)
    p2 = jnp.dot(h, wl_ref[...], preferred_element_type=jnp.float32)
    for q in range(2):
        t_refs[q][...] = p2[:, q * W2:(q + 1) * W2]
    r2_ref[...] = jnp.dot(h, wr_ref[...], preferred_element_type=jnp.float32) + b2_ref[...]


def _tc3_body(o0_ref, o1_ref, ca_ref, cb_ref, r2_ref, out_ref):
    cnt = ca_ref[...][:, 0:1] + cb_ref[...][:, 0:1]
    inv = 1.0 / jnp.maximum(cnt, 1.0)
    agg = jnp.concatenate([o0_ref[...], o1_ref[...]], axis=1)
    out_ref[...] = agg * inv + r2_ref[...]


def _tc1(x_pad, w1l_t, w1r_t, b1_row):
    blk = lambda i: (i, 0)
    full = lambda i: (0, 0)
    return pl.pallas_call(
        _tc1_body,
        grid=(NP // BN,),
        in_specs=[
            pl.BlockSpec((BN, D), blk),
            pl.BlockSpec((D, H), full),
            pl.BlockSpec((D, H), full),
            pl.BlockSpec((1, H), full),
        ],
        out_specs=[pl.BlockSpec((BN, W1), blk)] * 4 + [pl.BlockSpec((BN, H), blk)],
        out_shape=[jax.ShapeDtypeStruct((NP, W1), jnp.float32)] * 4
        + [jax.ShapeDtypeStruct((NP, H), jnp.float32)],
    )(x_pad, w1l_t, w1r_t, b1_row)


def _tc2(aggs, ca, cb, r1, w2l_t, w2r_t, b2_row):
    blk = lambda i: (i, 0)
    full = lambda i: (0, 0)
    return pl.pallas_call(
        _tc2_body,
        grid=(NP // BN,),
        in_specs=[pl.BlockSpec((BN, W1), blk)] * 4
        + [pl.BlockSpec((BN, 16), blk)] * 2
        + [
            pl.BlockSpec((BN, H), blk),
            pl.BlockSpec((H, C), full),
            pl.BlockSpec((H, C), full),
            pl.BlockSpec((1, C), full),
        ],
        out_specs=[pl.BlockSpec((BN, W2), blk)] * 2 + [pl.BlockSpec((BN, C), blk)],
        out_shape=[jax.ShapeDtypeStruct((NP, W2), jnp.float32)] * 2
        + [jax.ShapeDtypeStruct((NP, C), jnp.float32)],
    )(*aggs, ca, cb, r1, w2l_t, w2r_t, b2_row)


def _tc3(os, ca, cb, r2):
    bn3 = 400
    blk = lambda i: (i, 0)
    return pl.pallas_call(
        _tc3_body,
        grid=(N // bn3,),
        in_specs=[pl.BlockSpec((bn3, W2), blk)] * 2
        + [pl.BlockSpec((bn3, 16), blk)] * 2
        + [pl.BlockSpec((bn3, C), blk)],
        out_specs=pl.BlockSpec((bn3, C), blk),
        out_shape=jax.ShapeDtypeStruct((N, C), jnp.float32),
    )(*os, ca, cb, r2)


@functools.cache
def _make_sc_agg(w, groups, with_counts):
    passes = groups // NC
    mesh = plsc.VectorSubcoreMesh(
        core_axis_name="c", subcore_axis_name="s", num_cores=NC, num_subcores=NS)

    out_type = [jax.ShapeDtypeStruct((NP, w), jnp.float32) for _ in range(groups)]
    scratch = [
        pltpu.VMEM((CHUNKS, CH), jnp.int32),
        pltpu.VMEM((CHUNKS, CH), jnp.int32),
        pltpu.VMEM((CH, w), jnp.float32),
        pltpu.VMEM((CH, w), jnp.float32),
        pltpu.VMEM_SHARED((NP, w), jnp.float32),
        pltpu.SemaphoreType.DMA,
        pltpu.SemaphoreType.DMA,
    ]
    if with_counts:
        out_type += [jax.ShapeDtypeStruct((NP, 16), jnp.float32)] * 2
        scratch += [
            pltpu.VMEM((CH, 16), jnp.float32),
            pltpu.VMEM_SHARED((NP, 16), jnp.float32),
        ]

    @functools.partial(
        pl.kernel, out_type=tuple(out_type), mesh=mesh,
        scratch_types=tuple(scratch),
        compiler_params=pltpu.CompilerParams(use_tc_tiling_on_sc=False))
    def sc_agg(*args):
        tbls = args[:groups]
        srcix, dstix, zrow = args[groups:groups + 3]
        rest = args[groups + 3:]
        if with_counts:
            zcnt, ones_hbm = rest[0:2]
            rest = rest[2:]
        outs = rest[:groups]
        rest = rest[groups:]
        if with_counts:
            cnt_a, cnt_b = rest[0:2]
            src_v, dst_v, r0, r1, acc, s0, s1, ones_v, cacc = rest[2:]
        else:
            src_v, dst_v, r0, r1, acc, s0, s1 = rest
        c = lax.axis_index("c")
        s = lax.axis_index("s")
        rows = pl.ds(s * RPT, RPT)
        pltpu.sync_copy(srcix.at[pl.ds(s * CHUNKS, CHUNKS)], src_v)
        pltpu.sync_copy(dstix.at[pl.ds(s * CHUNKS, CHUNKS)], dst_v)
        if with_counts:
            pltpu.sync_copy(ones_hbm, ones_v)
            pltpu.sync_copy(zcnt, cacc.at[rows])

        def run_pass(tbl, do_counts):
            pltpu.async_copy(tbl.at[src_v.at[0]], r0, s0)

            def body(i, carry):
                j0 = 2 * i
                pltpu.make_async_copy(tbl.at[src_v.at[j0]], r0, s0).wait()
                pltpu.async_copy(tbl.at[src_v.at[j0 + 1]], r1, s1)
                pltpu.sync_copy(r0, acc.at[dst_v.at[j0]], add=True)
                if do_counts == 1:
                    pltpu.sync_copy(ones_v, cacc.at[dst_v.at[j0]], add=True)
                pltpu.make_async_copy(tbl.at[src_v.at[j0 + 1]], r1, s1).wait()

                @pl.when(i < CHUNKS // 2 - 1)
                def _():
                    pltpu.async_copy(tbl.at[src_v.at[j0 + 2]], r0, s0)

                pltpu.sync_copy(r1, acc.at[dst_v.at[j0 + 1]], add=True)
                if do_counts == 2:
                    pltpu.sync_copy(ones_v, cacc.at[dst_v.at[j0 + 1]], add=True)
                return carry

            lax.fori_loop(0, CHUNKS // 2, body, 0)

        for p in range(passes):
            pltpu.sync_copy(zrow, acc.at[rows])
            plsc.subcore_barrier()
            dc = with_counts and p == 0

            @pl.when(c == 0)
            def _():
                run_pass(tbls[p], 1 if dc else 0)

            @pl.when(c == 1)
            def _():
                run_pass(tbls[passes + p], 2 if dc else 0)

            plsc.subcore_barrier()

            out_c0 = outs[p]
            out_c1 = outs[passes + p]

            @pl.when(c == 0)
            def _():
                pltpu.sync_copy(acc.at[rows], out_c0.at[rows])

            @pl.when(c == 1)
            def _():
                pltpu.sync_copy(acc.at[rows], out_c1.at[rows])

        if with_counts:
            @pl.when(c == 0)
            def _():
                pltpu.sync_copy(cacc.at[rows], cnt_a.at[rows])

            @pl.when(c == 1)
            def _():
                pltpu.sync_copy(cacc.at[rows], cnt_b.at[rows])

    return sc_agg


def _sc_agg1(tbls, srcix, dstix, zrow, zcnt, ones_hbm):
    return _make_sc_agg(W1, 4, True)(*tbls, srcix, dstix, zrow, zcnt, ones_hbm)


def _sc_agg2(tbls, srcix, dstix, zrow):
    return _make_sc_agg(W2, 2, False)(*tbls, srcix, dstix, zrow)


def kernel(x, edge_index, W1_l, b1_l, W1_r, W2_l, b2_l, W2_r):
    x = x.astype(jnp.float32)
    src = edge_index[0].astype(jnp.int32)
    dst = edge_index[1].astype(jnp.int32)

    npad = EP - E
    pad_src = (lax.iota(jnp.int32, npad) * 37) % N
    pad_dst = N + lax.rem(lax.iota(jnp.int32, npad), NP - N)
    src_p = jnp.concatenate([src, pad_src])
    dst_p = jnp.concatenate([dst, pad_dst])

    src_ix = src_p.reshape(NS * CHUNKS, CH)
    dst_ix = dst_p.reshape(NS * CHUNKS, CH)

    x_pad = jnp.concatenate([x, jnp.zeros((NP - N, D), jnp.float32)])
    w1l_t = W1_l.T
    w1r_t = W1_r.T
    w2l_t = W2_l.T
    w2r_t = W2_r.T
    b1_row = b1_l.reshape(1, H)
    b2_row = b2_l.reshape(1, C)

    ones_rows = jnp.concatenate(
        [jnp.ones((CH, 1), jnp.float32), jnp.zeros((CH, 15), jnp.float32)], axis=1)
    zrow1 = jnp.zeros((RPT, W1), jnp.float32)
    zcnt = jnp.zeros((RPT, 16), jnp.float32)
    zrow2 = jnp.zeros((RPT, W2), jnp.float32)

    t1 = _tc1(x_pad, w1l_t, w1r_t, b1_row)
    r1 = t1[4]

    a0, a1, a2, a3, cnt_a, cnt_b = _sc_agg1(
        t1[:4], src_ix, dst_ix, zrow1, zcnt, ones_rows)

    t2 = _tc2((a0, a1, a2, a3), cnt_a, cnt_b, r1, w2l_t, w2r_t, b2_row)
    r2 = t2[2]

    o0, o1 = _sc_agg2(t2[:2], src_ix, dst_ix, zrow2)

    return _tc3((o0, o1), cnt_a, cnt_b, r2)


# speedup vs baseline: 1.0642x; 1.0636x over previous
import functools

import jax
import jax.numpy as jnp
from jax import lax
from jax.experimental import pallas as pl
from jax.experimental.pallas import tpu as pltpu
from jax.experimental.pallas import tpu_sc as plsc

N = 10000
NP = 10240
E = 160000
EP = 163840
D = 256
H = 256
C = 64
NC, NS = 2, 16
CH = 128
CHUNKS = EP // (NS * CH)
RPT = NP // NS
BN = 256
W1 = H // 4
W2 = C // 2


def _dot_t(a, w):
    return lax.dot_general(a, w, (((1,), (1,)), ((), ())),
                           preferred_element_type=jnp.float32)


def _tc1a_body(x_ref, wl_ref, *t_refs):
    p1 = _dot_t(x_ref[...], wl_ref[...])
    for q in range(4):
        t_refs[q][...] = p1[:, q * W1:(q + 1) * W1]


def _tc1b_body(x_ref, wr_ref, b1_ref, r1_ref):
    r1_ref[...] = _dot_t(x_ref[...], wr_ref[...]) + b1_ref[...]


def _tc2_body(a0_ref, a1_ref, a2_ref, a3_ref, ca_ref, cb_ref, r1_ref,
              wl_ref, wr_ref, b2_ref, *out_refs):
    t_refs, r2_ref = out_refs[:2], out_refs[2]
    cnt = ca_ref[...][:, 0:1] + cb_ref[...][:, 0:1]
    inv = 1.0 / jnp.maximum(cnt, 1.0)
    agg = jnp.concatenate(
        [a0_ref[...], a1_ref[...], a2_ref[...], a3_ref[...]], axis=1)
    h = jnp.maximum(agg * inv + r1_ref[...], 0.0)
    p2 = _dot_t(h, wl_ref[...])
    for q in range(2):
        t_refs[q][...] = p2[:, q * W2:(q + 1) * W2]
    r2_ref[...] = _dot_t(h, wr_ref[...]) + b2_ref[...]


def _tc3_body(o0_ref, o1_ref, ca_ref, cb_ref, r2_ref, out_ref):
    cnt = ca_ref[...][:, 0:1] + cb_ref[...][:, 0:1]
    inv = 1.0 / jnp.maximum(cnt, 1.0)
    agg = jnp.concatenate([o0_ref[...], o1_ref[...]], axis=1)
    out_ref[...] = agg * inv + r2_ref[...]


BN1 = 400


def _tc1a(x, w1l):
    blk = lambda i: (i, 0)
    full = lambda i: (0, 0)
    return pl.pallas_call(
        _tc1a_body,
        grid=(N // BN1,),
        in_specs=[
            pl.BlockSpec((BN1, D), blk),
            pl.BlockSpec((H, D), full),
        ],
        out_specs=[pl.BlockSpec((BN1, W1), blk)] * 4,
        out_shape=[jax.ShapeDtypeStruct((N, W1), jnp.float32)] * 4,
    )(x, w1l)


def _tc1b(x, w1r, b1_row):
    blk = lambda i: (i, 0)
    full = lambda i: (0, 0)
    return pl.pallas_call(
        _tc1b_body,
        grid=(N // BN1,),
        in_specs=[
            pl.BlockSpec((BN1, D), blk),
            pl.BlockSpec((H, D), full),
            pl.BlockSpec((1, H), full),
        ],
        out_specs=pl.BlockSpec((BN1, H), blk),
        out_shape=jax.ShapeDtypeStruct((N, H), jnp.float32),
    )(x, w1r, b1_row)


def _tc2(aggs, ca, cb, r1, w2l, w2r, b2_row):
    blk = lambda i: (i, 0)
    full = lambda i: (0, 0)
    return pl.pallas_call(
        _tc2_body,
        grid=(N // BN1,),
        in_specs=[pl.BlockSpec((BN1, W1), blk)] * 4
        + [pl.BlockSpec((BN1, 16), blk)] * 2
        + [
            pl.BlockSpec((BN1, H), blk),
            pl.BlockSpec((C, H), full),
            pl.BlockSpec((C, H), full),
            pl.BlockSpec((1, C), full),
        ],
        out_specs=[pl.BlockSpec((BN1, W2), blk)] * 2 + [pl.BlockSpec((BN1, C), blk)],
        out_shape=[jax.ShapeDtypeStruct((N, W2), jnp.float32)] * 2
        + [jax.ShapeDtypeStruct((N, C), jnp.float32)],
    )(*aggs, ca, cb, r1, w2l, w2r, b2_row)


def _tc3(os, ca, cb, r2):
    bn3 = 400
    blk = lambda i: (i, 0)
    return pl.pallas_call(
        _tc3_body,
        grid=(N // bn3,),
        in_specs=[pl.BlockSpec((bn3, W2), blk)] * 2
        + [pl.BlockSpec((bn3, 16), blk)] * 2
        + [pl.BlockSpec((bn3, C), blk)],
        out_specs=pl.BlockSpec((bn3, C), blk),
        out_shape=jax.ShapeDtypeStruct((N, C), jnp.float32),
    )(*os, ca, cb, r2)


@functools.cache
def _make_sc_agg(w, groups, with_counts):
    passes = groups // NC
    mesh = plsc.VectorSubcoreMesh(
        core_axis_name="c", subcore_axis_name="s", num_cores=NC, num_subcores=NS)

    out_type = [jax.ShapeDtypeStruct((NP, w), jnp.float32) for _ in range(groups)]
    scratch = [
        pltpu.VMEM((CHUNKS, CH), jnp.int32),
        pltpu.VMEM((CHUNKS, CH), jnp.int32),
        pltpu.VMEM((CH, w), jnp.float32),
        pltpu.VMEM((CH, w), jnp.float32),
        pltpu.VMEM_SHARED((NP, w), jnp.float32),
        pltpu.SemaphoreType.DMA,
        pltpu.SemaphoreType.DMA,
    ]
    if with_counts:
        out_type += [jax.ShapeDtypeStruct((NP, 16), jnp.float32)] * 2
        scratch += [
            pltpu.VMEM((CH, 16), jnp.float32),
            pltpu.VMEM_SHARED((NP, 16), jnp.float32),
        ]

    @functools.partial(
        pl.kernel, out_type=tuple(out_type), mesh=mesh,
        scratch_types=tuple(scratch),
        compiler_params=pltpu.CompilerParams(use_tc_tiling_on_sc=False))
    def sc_agg(*args):
        tbls = args[:groups]
        srcix, dstix, zrow = args[groups:groups + 3]
        rest = args[groups + 3:]
        if with_counts:
            zcnt, ones_hbm = rest[0:2]
            rest = rest[2:]
        outs = rest[:groups]
        rest = rest[groups:]
        if with_counts:
            cnt_a, cnt_b = rest[0:2]
            src_v, dst_v, r0, r1, acc, s0, s1, ones_v, cacc = rest[2:]
        else:
            src_v, dst_v, r0, r1, acc, s0, s1 = rest
        c = lax.axis_index("c")
        s = lax.axis_index("s")
        rows = pl.ds(s * RPT, RPT)
        pltpu.sync_copy(srcix.at[pl.ds(s * CHUNKS, CHUNKS)], src_v)
        pltpu.sync_copy(dstix.at[pl.ds(s * CHUNKS, CHUNKS)], dst_v)
        if with_counts:
            pltpu.sync_copy(ones_hbm, ones_v)
            pltpu.sync_copy(zcnt, cacc.at[rows])

        def run_pass(tbl, do_counts):
            pltpu.async_copy(tbl.at[src_v.at[0]], r0, s0)

            def body(i, carry):
                j0 = 2 * i
                pltpu.make_async_copy(tbl.at[src_v.at[j0]], r0, s0).wait()
                pltpu.async_copy(tbl.at[src_v.at[j0 + 1]], r1, s1)
                pltpu.sync_copy(r0, acc.at[dst_v.at[j0]], add=True)
                if do_counts == 1:
                    pltpu.sync_copy(ones_v, cacc.at[dst_v.at[j0]], add=True)
                pltpu.make_async_copy(tbl.at[src_v.at[j0 + 1]], r1, s1).wait()

                @pl.when(i < CHUNKS // 2 - 1)
                def _():
                    pltpu.async_copy(tbl.at[src_v.at[j0 + 2]], r0, s0)

                pltpu.sync_copy(r1, acc.at[dst_v.at[j0 + 1]], add=True)
                if do_counts == 2:
                    pltpu.sync_copy(ones_v, cacc.at[dst_v.at[j0 + 1]], add=True)
                return carry

            lax.fori_loop(0, CHUNKS // 2, body, 0)

        for p in range(passes):
            pltpu.sync_copy(zrow, acc.at[rows])
            plsc.subcore_barrier()
            dc = with_counts and p == 0

            @pl.when(c == 0)
            def _():
                run_pass(tbls[p], 1 if dc else 0)

            @pl.when(c == 1)
            def _():
                run_pass(tbls[passes + p], 2 if dc else 0)

            plsc.subcore_barrier()

            out_c0 = outs[p]
            out_c1 = outs[passes + p]

            @pl.when(c == 0)
            def _():
                pltpu.sync_copy(acc.at[rows], out_c0.at[rows])

            @pl.when(c == 1)
            def _():
                pltpu.sync_copy(acc.at[rows], out_c1.at[rows])

        if with_counts:
            @pl.when(c == 0)
            def _():
                pltpu.sync_copy(cacc.at[rows], cnt_a.at[rows])

            @pl.when(c == 1)
            def _():
                pltpu.sync_copy(cacc.at[rows], cnt_b.at[rows])

    return sc_agg


def _sc_agg1(tbls, srcix, dstix, zrow, zcnt, ones_hbm):
    return _make_sc_agg(W1, 4, True)(*tbls, srcix, dstix, zrow, zcnt, ones_hbm)


def _sc_agg2(tbls, srcix, dstix, zrow):
    return _make_sc_agg(W2, 2, False)(*tbls, srcix, dstix, zrow)


def kernel(x, edge_index, W1_l, b1_l, W1_r, W2_l, b2_l, W2_r):
    x = x.astype(jnp.float32)
    src = edge_index[0].astype(jnp.int32)
    dst = edge_index[1].astype(jnp.int32)

    npad = EP - E
    pad_src = (lax.iota(jnp.int32, npad) * 37) % N
    pad_dst = N + lax.rem(lax.iota(jnp.int32, npad), NP - N)
    src_p = jnp.concatenate([src, pad_src])
    dst_p = jnp.concatenate([dst, pad_dst])

    src_ix = src_p.reshape(NS * CHUNKS, CH)
    dst_ix = dst_p.reshape(NS * CHUNKS, CH)

    b1_row = b1_l.reshape(1, H)
    b2_row = b2_l.reshape(1, C)

    ones_rows = jnp.concatenate(
        [jnp.ones((CH, 1), jnp.float32), jnp.zeros((CH, 15), jnp.float32)], axis=1)
    zrow1 = jnp.zeros((RPT, W1), jnp.float32)
    zcnt = jnp.zeros((RPT, 16), jnp.float32)
    zrow2 = jnp.zeros((RPT, W2), jnp.float32)

    t1 = _tc1a(x, W1_l)
    a0, a1, a2, a3, cnt_a, cnt_b = _sc_agg1(
        t1, src_ix, dst_ix, zrow1, zcnt, ones_rows)
    r1 = _tc1b(x, W1_r, b1_row)

    t2 = _tc2((a0, a1, a2, a3), cnt_a, cnt_b, r1, W2_l, W2_r, b2_row)
    r2 = t2[2]

    o0, o1 = _sc_agg2(t2[:2], src_ix, dst_ix, zrow2)

    return _tc3((o0, o1), cnt_a, cnt_b, r2)


# speedup vs baseline: 1.1303x; 1.0620x over previous
import functools

import jax
import jax.numpy as jnp
from jax import lax
from jax.experimental import pallas as pl
from jax.experimental.pallas import tpu as pltpu
from jax.experimental.pallas import tpu_sc as plsc

N = 10000
NP = 10240
E = 160000
EP = 163840
D = 256
H = 256
C = 64
NC, NS = 2, 16
CH = 128
CHUNKS = EP // (NS * CH)
RPT = NP // NS
BN = 256
W1 = H // 4
W2 = C // 2


def _dot_t(a, w):
    return lax.dot_general(a, w, (((1,), (1,)), ((), ())),
                           preferred_element_type=jnp.float32)


def _tc1a_body(x_ref, wl_ref, *t_refs):
    p1 = _dot_t(x_ref[...], wl_ref[...])
    for q in range(4):
        t_refs[q][...] = p1[:, q * W1:(q + 1) * W1]


def _tc1b_body(x_ref, wr_ref, b1_ref, r1_ref):
    r1_ref[...] = _dot_t(x_ref[...], wr_ref[...]) + b1_ref[...]


def _tc2_body(a0_ref, a1_ref, a2_ref, a3_ref, ca_ref, cb_ref, r1_ref,
              wl_ref, wr_ref, b2_ref, *out_refs):
    t2_ref, r2_ref = out_refs
    cnt = ca_ref[...][:, 0:1] + cb_ref[...][:, 0:1]
    inv = 1.0 / jnp.maximum(cnt, 1.0)
    agg = jnp.concatenate(
        [a0_ref[...], a1_ref[...], a2_ref[...], a3_ref[...]], axis=1)
    h = jnp.maximum(agg * inv + r1_ref[...], 0.0)
    t2_ref[...] = _dot_t(h, wl_ref[...])
    r2_ref[...] = _dot_t(h, wr_ref[...]) + b2_ref[...]


def _tc3_body(o0_ref, o1_ref, ca_ref, cb_ref, r2_ref, out_ref):
    cnt = ca_ref[...][:, 0:1] + cb_ref[...][:, 0:1]
    inv = 1.0 / jnp.maximum(cnt, 1.0)
    out_ref[...] = (o0_ref[...] + o1_ref[...]) * inv + r2_ref[...]


BN1 = 400


def _tc1a(x, w1l):
    blk = lambda i: (i, 0)
    full = lambda i: (0, 0)
    return pl.pallas_call(
        _tc1a_body,
        grid=(N // BN1,),
        in_specs=[
            pl.BlockSpec((BN1, D), blk),
            pl.BlockSpec((H, D), full),
        ],
        out_specs=[pl.BlockSpec((BN1, W1), blk)] * 4,
        out_shape=[jax.ShapeDtypeStruct((N, W1), jnp.float32)] * 4,
    )(x, w1l)


def _tc1b(x, w1r, b1_row):
    blk = lambda i: (i, 0)
    full = lambda i: (0, 0)
    return pl.pallas_call(
        _tc1b_body,
        grid=(N // BN1,),
        in_specs=[
            pl.BlockSpec((BN1, D), blk),
            pl.BlockSpec((H, D), full),
            pl.BlockSpec((1, H), full),
        ],
        out_specs=pl.BlockSpec((BN1, H), blk),
        out_shape=jax.ShapeDtypeStruct((N, H), jnp.float32),
    )(x, w1r, b1_row)


def _tc2(aggs, ca, cb, r1, w2l, w2r, b2_row):
    blk = lambda i: (i, 0)
    full = lambda i: (0, 0)
    return pl.pallas_call(
        _tc2_body,
        grid=(N // BN1,),
        in_specs=[pl.BlockSpec((BN1, W1), blk)] * 4
        + [pl.BlockSpec((BN1, 16), blk)] * 2
        + [
            pl.BlockSpec((BN1, H), blk),
            pl.BlockSpec((C, H), full),
            pl.BlockSpec((C, H), full),
            pl.BlockSpec((1, C), full),
        ],
        out_specs=[pl.BlockSpec((BN1, C), blk)] * 2,
        out_shape=[jax.ShapeDtypeStruct((N, C), jnp.float32)] * 2,
    )(*aggs, ca, cb, r1, w2l, w2r, b2_row)


def _tc3(os, ca, cb, r2):
    bn3 = 400
    blk = lambda i: (i, 0)
    return pl.pallas_call(
        _tc3_body,
        grid=(N // bn3,),
        in_specs=[pl.BlockSpec((bn3, C), blk)] * 2
        + [pl.BlockSpec((bn3, 16), blk)] * 2
        + [pl.BlockSpec((bn3, C), blk)],
        out_specs=pl.BlockSpec((bn3, C), blk),
        out_shape=jax.ShapeDtypeStruct((N, C), jnp.float32),
    )(*os, ca, cb, r2)


@functools.cache
def _make_sc_agg(w, groups, with_counts):
    passes = groups // NC
    mesh = plsc.VectorSubcoreMesh(
        core_axis_name="c", subcore_axis_name="s", num_cores=NC, num_subcores=NS)

    out_type = [jax.ShapeDtypeStruct((NP, w), jnp.float32) for _ in range(groups)]
    scratch = [
        pltpu.VMEM((CHUNKS, CH), jnp.int32),
        pltpu.VMEM((CHUNKS, CH), jnp.int32),
        pltpu.VMEM((CH, w), jnp.float32),
        pltpu.VMEM((CH, w), jnp.float32),
        pltpu.VMEM_SHARED((NP, w), jnp.float32),
        pltpu.SemaphoreType.DMA,
        pltpu.SemaphoreType.DMA,
    ]
    if with_counts:
        out_type += [jax.ShapeDtypeStruct((NP, 16), jnp.float32)] * 2
        scratch += [
            pltpu.VMEM((CH, 16), jnp.float32),
            pltpu.VMEM_SHARED((NP, 16), jnp.float32),
        ]

    @functools.partial(
        pl.kernel, out_type=tuple(out_type), mesh=mesh,
        scratch_types=tuple(scratch),
        compiler_params=pltpu.CompilerParams(use_tc_tiling_on_sc=False))
    def sc_agg(*args):
        tbls = args[:groups]
        srcix, dstix, zrow = args[groups:groups + 3]
        rest = args[groups + 3:]
        if with_counts:
            zcnt, ones_hbm = rest[0:2]
            rest = rest[2:]
        outs = rest[:groups]
        rest = rest[groups:]
        if with_counts:
            cnt_a, cnt_b = rest[0:2]
            src_v, dst_v, r0, r1, acc, s0, s1, ones_v, cacc = rest[2:]
        else:
            src_v, dst_v, r0, r1, acc, s0, s1 = rest
        c = lax.axis_index("c")
        s = lax.axis_index("s")
        rows = pl.ds(s * RPT, RPT)
        pltpu.sync_copy(srcix.at[pl.ds(s * CHUNKS, CHUNKS)], src_v)
        pltpu.sync_copy(dstix.at[pl.ds(s * CHUNKS, CHUNKS)], dst_v)
        if with_counts:
            pltpu.sync_copy(ones_hbm, ones_v)
            pltpu.sync_copy(zcnt, cacc.at[rows])

        def run_pass(tbl, do_counts):
            pltpu.async_copy(tbl.at[src_v.at[0]], r0, s0)

            def body(i, carry):
                j0 = 2 * i
                pltpu.make_async_copy(tbl.at[src_v.at[j0]], r0, s0).wait()
                pltpu.async_copy(tbl.at[src_v.at[j0 + 1]], r1, s1)
                pltpu.sync_copy(r0, acc.at[dst_v.at[j0]], add=True)
                if do_counts == 1:
                    pltpu.sync_copy(ones_v, cacc.at[dst_v.at[j0]], add=True)
                pltpu.make_async_copy(tbl.at[src_v.at[j0 + 1]], r1, s1).wait()

                @pl.when(i < CHUNKS // 2 - 1)
                def _():
                    pltpu.async_copy(tbl.at[src_v.at[j0 + 2]], r0, s0)

                pltpu.sync_copy(r1, acc.at[dst_v.at[j0 + 1]], add=True)
                if do_counts == 2:
                    pltpu.sync_copy(ones_v, cacc.at[dst_v.at[j0 + 1]], add=True)
                return carry

            lax.fori_loop(0, CHUNKS // 2, body, 0)

        for p in range(passes):
            pltpu.sync_copy(zrow, acc.at[rows])
            plsc.subcore_barrier()
            dc = with_counts and p == 0

            @pl.when(c == 0)
            def _():
                run_pass(tbls[p], 1 if dc else 0)

            @pl.when(c == 1)
            def _():
                run_pass(tbls[passes + p], 2 if dc else 0)

            plsc.subcore_barrier()

            out_c0 = outs[p]
            out_c1 = outs[passes + p]

            @pl.when(c == 0)
            def _():
                pltpu.sync_copy(acc.at[rows], out_c0.at[rows])

            @pl.when(c == 1)
            def _():
                pltpu.sync_copy(acc.at[rows], out_c1.at[rows])

        if with_counts:
            @pl.when(c == 0)
            def _():
                pltpu.sync_copy(cacc.at[rows], cnt_a.at[rows])

            @pl.when(c == 1)
            def _():
                pltpu.sync_copy(cacc.at[rows], cnt_b.at[rows])

    return sc_agg


def _sc_agg1(tbls, srcix, dstix, zrow, zcnt, ones_hbm):
    return _make_sc_agg(W1, 4, True)(*tbls, srcix, dstix, zrow, zcnt, ones_hbm)


CH2 = CHUNKS // NC


@functools.cache
def _make_sc_agg_edge():
    mesh = plsc.VectorSubcoreMesh(
        core_axis_name="c", subcore_axis_name="s", num_cores=NC, num_subcores=NS)

    @functools.partial(
        pl.kernel,
        out_type=(jax.ShapeDtypeStruct((NP, C), jnp.float32),
                  jax.ShapeDtypeStruct((NP, C), jnp.float32)),
        mesh=mesh,
        scratch_types=(
            pltpu.VMEM((CH2, CH), jnp.int32),
            pltpu.VMEM((CH2, CH), jnp.int32),
            pltpu.VMEM((CH, C), jnp.float32),
            pltpu.VMEM((CH, C), jnp.float32),
            pltpu.VMEM_SHARED((NP, C), jnp.float32),
            pltpu.SemaphoreType.DMA,
            pltpu.SemaphoreType.DMA,
        ),
        compiler_params=pltpu.CompilerParams(use_tc_tiling_on_sc=False))
    def sc_agg_edge(tbl, srcix, dstix, zrow, o_a, o_b,
                    src_v, dst_v, r0, r1, acc, s0, s1):
        c = lax.axis_index("c")
        s = lax.axis_index("s")
        w = c * NS + s
        rows = pl.ds(s * RPT, RPT)
        pltpu.sync_copy(srcix.at[pl.ds(w * CH2, CH2)], src_v)
        pltpu.sync_copy(dstix.at[pl.ds(w * CH2, CH2)], dst_v)
        pltpu.sync_copy(zrow, acc.at[rows])
        plsc.subcore_barrier()

        pltpu.async_copy(tbl.at[src_v.at[0]], r0, s0)

        def body(i, carry):
            j0 = 2 * i
            pltpu.make_async_copy(tbl.at[src_v.at[j0]], r0, s0).wait()
            pltpu.async_copy(tbl.at[src_v.at[j0 + 1]], r1, s1)
            pltpu.sync_copy(r0, acc.at[dst_v.at[j0]], add=True)
            pltpu.make_async_copy(tbl.at[src_v.at[j0 + 1]], r1, s1).wait()

            @pl.when(i < CH2 // 2 - 1)
            def _():
                pltpu.async_copy(tbl.at[src_v.at[j0 + 2]], r0, s0)

            pltpu.sync_copy(r1, acc.at[dst_v.at[j0 + 1]], add=True)
            return carry

        lax.fori_loop(0, CH2 // 2, body, 0)
        plsc.subcore_barrier()

        @pl.when(c == 0)
        def _():
            pltpu.sync_copy(acc.at[rows], o_a.at[rows])

        @pl.when(c == 1)
        def _():
            pltpu.sync_copy(acc.at[rows], o_b.at[rows])

    return sc_agg_edge


def _sc_agg2(tbl, srcix, dstix, zrow):
    return _make_sc_agg_edge()(tbl, srcix, dstix, zrow)


def kernel(x, edge_index, W1_l, b1_l, W1_r, W2_l, b2_l, W2_r):
    x = x.astype(jnp.float32)
    src = edge_index[0].astype(jnp.int32)
    dst = edge_index[1].astype(jnp.int32)

    npad = EP - E
    pad_src = (lax.iota(jnp.int32, npad) * 37) % N
    pad_dst = N + lax.rem(lax.iota(jnp.int32, npad), NP - N)
    src_p = jnp.concatenate([src, pad_src])
    dst_p = jnp.concatenate([dst, pad_dst])

    src_ix = src_p.reshape(NS * CHUNKS, CH)
    dst_ix = dst_p.reshape(NS * CHUNKS, CH)

    b1_row = b1_l.reshape(1, H)
    b2_row = b2_l.reshape(1, C)

    ones_rows = jnp.concatenate(
        [jnp.ones((CH, 1), jnp.float32), jnp.zeros((CH, 15), jnp.float32)], axis=1)
    zrow1 = jnp.zeros((RPT, W1), jnp.float32)
    zcnt = jnp.zeros((RPT, 16), jnp.float32)
    zrow2 = jnp.zeros((RPT, C), jnp.float32)

    t1 = _tc1a(x, W1_l)
    a0, a1, a2, a3, cnt_a, cnt_b = _sc_agg1(
        t1, src_ix, dst_ix, zrow1, zcnt, ones_rows)
    r1 = _tc1b(x, W1_r, b1_row)

    t2_tbl, r2 = _tc2((a0, a1, a2, a3), cnt_a, cnt_b, r1, W2_l, W2_r, b2_row)

    o0, o1 = _sc_agg2(t2_tbl, src_ix, dst_ix, zrow2)

    return _tc3((o0, o1), cnt_a, cnt_b, r2)


# speedup vs baseline: 1.1634x; 1.0293x over previous
import functools

import jax
import jax.numpy as jnp
from jax import lax
from jax.experimental import pallas as pl
from jax.experimental.pallas import tpu as pltpu
from jax.experimental.pallas import tpu_sc as plsc

N = 10000
NP = 10240
E = 160000
EP = 163840
D = 256
H = 256
C = 64
NC, NS = 2, 16
CH = 128
CHUNKS = EP // (NS * CH)
RPT = NP // NS
BN = 256
W1 = H // 4
W2 = C // 2


def _dot_t(a, w):
    return lax.dot_general(a, w, (((1,), (1,)), ((), ())),
                           preferred_element_type=jnp.float32)


def _tc1a_body(x_ref, wl_ref, *t_refs):
    p1 = _dot_t(x_ref[...], wl_ref[...])
    for q in range(4):
        t_refs[q][...] = p1[:, q * W1:(q + 1) * W1]


def _tc1b_body(x_ref, wr_ref, b1_ref, r1_ref):
    r1_ref[...] = _dot_t(x_ref[...], wr_ref[...]) + b1_ref[...]


def _hidden(a0_ref, a1_ref, a2_ref, a3_ref, ca_ref, cb_ref, r1_ref):
    cnt = ca_ref[...][:, 0:1] + cb_ref[...][:, 0:1]
    inv = 1.0 / jnp.maximum(cnt, 1.0)
    agg = jnp.concatenate(
        [a0_ref[...], a1_ref[...], a2_ref[...], a3_ref[...]], axis=1)
    return jnp.maximum(agg * inv + r1_ref[...], 0.0)


def _tc2a_body(a0_ref, a1_ref, a2_ref, a3_ref, ca_ref, cb_ref, r1_ref,
               wl_ref, t2_ref):
    h = _hidden(a0_ref, a1_ref, a2_ref, a3_ref, ca_ref, cb_ref, r1_ref)
    t2_ref[...] = _dot_t(h, wl_ref[...])


def _tc2b_body(a0_ref, a1_ref, a2_ref, a3_ref, ca_ref, cb_ref, r1_ref,
               wr_ref, b2_ref, r2_ref):
    h = _hidden(a0_ref, a1_ref, a2_ref, a3_ref, ca_ref, cb_ref, r1_ref)
    r2_ref[...] = _dot_t(h, wr_ref[...]) + b2_ref[...]


def _tc3_body(o0_ref, o1_ref, ca_ref, cb_ref, r2_ref, out_ref):
    cnt = ca_ref[...][:, 0:1] + cb_ref[...][:, 0:1]
    inv = 1.0 / jnp.maximum(cnt, 1.0)
    out_ref[...] = (o0_ref[...] + o1_ref[...]) * inv + r2_ref[...]


BN1 = 1000


def _tc1a(x, w1l):
    blk = lambda i: (i, 0)
    full = lambda i: (0, 0)
    return pl.pallas_call(
        _tc1a_body,
        grid=(N // BN1,),
        in_specs=[
            pl.BlockSpec((BN1, D), blk),
            pl.BlockSpec((H, D), full),
        ],
        out_specs=[pl.BlockSpec((BN1, W1), blk)] * 4,
        out_shape=[jax.ShapeDtypeStruct((N, W1), jnp.float32)] * 4,
    )(x, w1l)


def _tc1b(x, w1r, b1_row):
    blk = lambda i: (i, 0)
    full = lambda i: (0, 0)
    return pl.pallas_call(
        _tc1b_body,
        grid=(N // BN1,),
        in_specs=[
            pl.BlockSpec((BN1, D), blk),
            pl.BlockSpec((H, D), full),
            pl.BlockSpec((1, H), full),
        ],
        out_specs=pl.BlockSpec((BN1, H), blk),
        out_shape=jax.ShapeDtypeStruct((N, H), jnp.float32),
    )(x, w1r, b1_row)


def _tc2a(aggs, ca, cb, r1, w2l):
    blk = lambda i: (i, 0)
    full = lambda i: (0, 0)
    return pl.pallas_call(
        _tc2a_body,
        grid=(N // BN1,),
        in_specs=[pl.BlockSpec((BN1, W1), blk)] * 4
        + [pl.BlockSpec((BN1, 16), blk)] * 2
        + [
            pl.BlockSpec((BN1, H), blk),
            pl.BlockSpec((C, H), full),
        ],
        out_specs=pl.BlockSpec((BN1, C), blk),
        out_shape=jax.ShapeDtypeStruct((N, C), jnp.float32),
    )(*aggs, ca, cb, r1, w2l)


def _tc2b(aggs, ca, cb, r1, w2r, b2_row):
    blk = lambda i: (i, 0)
    full = lambda i: (0, 0)
    return pl.pallas_call(
        _tc2b_body,
        grid=(N // BN1,),
        in_specs=[pl.BlockSpec((BN1, W1), blk)] * 4
        + [pl.BlockSpec((BN1, 16), blk)] * 2
        + [
            pl.BlockSpec((BN1, H), blk),
            pl.BlockSpec((C, H), full),
            pl.BlockSpec((1, C), full),
        ],
        out_specs=pl.BlockSpec((BN1, C), blk),
        out_shape=jax.ShapeDtypeStruct((N, C), jnp.float32),
    )(*aggs, ca, cb, r1, w2r, b2_row)


def _tc3(os, ca, cb, r2):
    bn3 = 400
    blk = lambda i: (i, 0)
    return pl.pallas_call(
        _tc3_body,
        grid=(N // bn3,),
        in_specs=[pl.BlockSpec((bn3, C), blk)] * 2
        + [pl.BlockSpec((bn3, 16), blk)] * 2
        + [pl.BlockSpec((bn3, C), blk)],
        out_specs=pl.BlockSpec((bn3, C), blk),
        out_shape=jax.ShapeDtypeStruct((N, C), jnp.float32),
    )(*os, ca, cb, r2)


@functools.cache
def _make_sc_agg(w, groups, with_counts):
    passes = groups // NC
    mesh = plsc.VectorSubcoreMesh(
        core_axis_name="c", subcore_axis_name="s", num_cores=NC, num_subcores=NS)

    out_type = [jax.ShapeDtypeStruct((NP, w), jnp.float32) for _ in range(groups)]
    scratch = [
        pltpu.VMEM((CHUNKS, CH), jnp.int32),
        pltpu.VMEM((CHUNKS, CH), jnp.int32),
        pltpu.VMEM((CH, w), jnp.float32),
        pltpu.VMEM((CH, w), jnp.float32),
        pltpu.VMEM_SHARED((NP, w), jnp.float32),
        pltpu.SemaphoreType.DMA,
        pltpu.SemaphoreType.DMA,
    ]
    if with_counts:
        out_type += [jax.ShapeDtypeStruct((NP, 16), jnp.float32)] * 2
        scratch += [
            pltpu.VMEM((CH, 16), jnp.float32),
            pltpu.VMEM_SHARED((NP, 16), jnp.float32),
        ]

    @functools.partial(
        pl.kernel, out_type=tuple(out_type), mesh=mesh,
        scratch_types=tuple(scratch),
        compiler_params=pltpu.CompilerParams(use_tc_tiling_on_sc=False))
    def sc_agg(*args):
        tbls = args[:groups]
        srcix, dstix, zrow = args[groups:groups + 3]
        rest = args[groups + 3:]
        if with_counts:
            zcnt, ones_hbm = rest[0:2]
            rest = rest[2:]
        outs = rest[:groups]
        rest = rest[groups:]
        if with_counts:
            cnt_a, cnt_b = rest[0:2]
            src_v, dst_v, r0, r1, acc, s0, s1, ones_v, cacc = rest[2:]
        else:
            src_v, dst_v, r0, r1, acc, s0, s1 = rest
        c = lax.axis_index("c")
        s = lax.axis_index("s")
        rows = pl.ds(s * RPT, RPT)
        pltpu.sync_copy(srcix.at[pl.ds(s * CHUNKS, CHUNKS)], src_v)
        pltpu.sync_copy(dstix.at[pl.ds(s * CHUNKS, CHUNKS)], dst_v)
        if with_counts:
            pltpu.sync_copy(ones_hbm, ones_v)
            pltpu.sync_copy(zcnt, cacc.at[rows])

        def run_pass(tbl, do_counts):
            pltpu.async_copy(tbl.at[src_v.at[0]], r0, s0)

            def body(i, carry):
                j0 = 2 * i
                pltpu.make_async_copy(tbl.at[src_v.at[j0]], r0, s0).wait()
                pltpu.async_copy(tbl.at[src_v.at[j0 + 1]], r1, s1)
                pltpu.sync_copy(r0, acc.at[dst_v.at[j0]], add=True)
                if do_counts == 1:
                    pltpu.sync_copy(ones_v, cacc.at[dst_v.at[j0]], add=True)
                pltpu.make_async_copy(tbl.at[src_v.at[j0 + 1]], r1, s1).wait()

                @pl.when(i < CHUNKS // 2 - 1)
                def _():
                    pltpu.async_copy(tbl.at[src_v.at[j0 + 2]], r0, s0)

                pltpu.sync_copy(r1, acc.at[dst_v.at[j0 + 1]], add=True)
                if do_counts == 2:
                    pltpu.sync_copy(ones_v, cacc.at[dst_v.at[j0 + 1]], add=True)
                return carry

            lax.fori_loop(0, CHUNKS // 2, body, 0)

        for p in range(passes):
            pltpu.sync_copy(zrow, acc.at[rows])
            plsc.subcore_barrier()
            dc = with_counts and p == 0

            @pl.when(c == 0)
            def _():
                run_pass(tbls[p], 1 if dc else 0)

            @pl.when(c == 1)
            def _():
                run_pass(tbls[passes + p], 2 if dc else 0)

            plsc.subcore_barrier()

            out_c0 = outs[p]
            out_c1 = outs[passes + p]

            @pl.when(c == 0)
            def _():
                pltpu.sync_copy(acc.at[rows], out_c0.at[rows])

            @pl.when(c == 1)
            def _():
                pltpu.sync_copy(acc.at[rows], out_c1.at[rows])

        if with_counts:
            @pl.when(c == 0)
            def _():
                pltpu.sync_copy(cacc.at[rows], cnt_a.at[rows])

            @pl.when(c == 1)
            def _():
                pltpu.sync_copy(cacc.at[rows], cnt_b.at[rows])

    return sc_agg


def _sc_agg1(tbls, srcix, dstix, zrow, zcnt, ones_hbm):
    return _make_sc_agg(W1, 4, True)(*tbls, srcix, dstix, zrow, zcnt, ones_hbm)


CH2 = CHUNKS // NC


@functools.cache
def _make_sc_agg_edge():
    mesh = plsc.VectorSubcoreMesh(
        core_axis_name="c", subcore_axis_name="s", num_cores=NC, num_subcores=NS)

    @functools.partial(
        pl.kernel,
        out_type=(jax.ShapeDtypeStruct((NP, C), jnp.float32),
                  jax.ShapeDtypeStruct((NP, C), jnp.float32)),
        mesh=mesh,
        scratch_types=(
            pltpu.VMEM((CH2, CH), jnp.int32),
            pltpu.VMEM((CH2, CH), jnp.int32),
            pltpu.VMEM((CH, C), jnp.float32),
            pltpu.VMEM((CH, C), jnp.float32),
            pltpu.VMEM_SHARED((NP, C), jnp.float32),
            pltpu.SemaphoreType.DMA,
            pltpu.SemaphoreType.DMA,
        ),
        compiler_params=pltpu.CompilerParams(use_tc_tiling_on_sc=False))
    def sc_agg_edge(tbl, srcix, dstix, zrow, o_a, o_b,
                    src_v, dst_v, r0, r1, acc, s0, s1):
        c = lax.axis_index("c")
        s = lax.axis_index("s")
        w = c * NS + s
        rows = pl.ds(s * RPT, RPT)
        pltpu.sync_copy(srcix.at[pl.ds(w * CH2, CH2)], src_v)
        pltpu.sync_copy(dstix.at[pl.ds(w * CH2, CH2)], dst_v)
        pltpu.sync_copy(zrow, acc.at[rows])
        plsc.subcore_barrier()

        pltpu.async_copy(tbl.at[src_v.at[0]], r0, s0)

        def body(i, carry):
            j0 = 2 * i
            pltpu.make_async_copy(tbl.at[src_v.at[j0]], r0, s0).wait()
            pltpu.async_copy(tbl.at[src_v.at[j0 + 1]], r1, s1)
            pltpu.sync_copy(r0, acc.at[dst_v.at[j0]], add=True)
            pltpu.make_async_copy(tbl.at[src_v.at[j0 + 1]], r1, s1).wait()

            @pl.when(i < CH2 // 2 - 1)
            def _():
                pltpu.async_copy(tbl.at[src_v.at[j0 + 2]], r0, s0)

            pltpu.sync_copy(r1, acc.at[dst_v.at[j0 + 1]], add=True)
            return carry

        lax.fori_loop(0, CH2 // 2, body, 0)
        plsc.subcore_barrier()

        @pl.when(c == 0)
        def _():
            pltpu.sync_copy(acc.at[rows], o_a.at[rows])

        @pl.when(c == 1)
        def _():
            pltpu.sync_copy(acc.at[rows], o_b.at[rows])

    return sc_agg_edge


def _sc_agg2(tbl, srcix, dstix, zrow):
    return _make_sc_agg_edge()(tbl, srcix, dstix, zrow)


def kernel(x, edge_index, W1_l, b1_l, W1_r, W2_l, b2_l, W2_r):
    x = x.astype(jnp.float32)
    src = edge_index[0].astype(jnp.int32)
    dst = edge_index[1].astype(jnp.int32)

    npad = EP - E
    pad_src = (lax.iota(jnp.int32, npad) * 37) % N
    pad_dst = N + lax.rem(lax.iota(jnp.int32, npad), NP - N)
    src_p = jnp.concatenate([src, pad_src])
    dst_p = jnp.concatenate([dst, pad_dst])

    src_ix = src_p.reshape(NS * CHUNKS, CH)
    dst_ix = dst_p.reshape(NS * CHUNKS, CH)

    b1_row = b1_l.reshape(1, H)
    b2_row = b2_l.reshape(1, C)

    ones_rows = jnp.concatenate(
        [jnp.ones((CH, 1), jnp.float32), jnp.zeros((CH, 15), jnp.float32)], axis=1)
    zrow1 = jnp.zeros((RPT, W1), jnp.float32)
    zcnt = jnp.zeros((RPT, 16), jnp.float32)
    zrow2 = jnp.zeros((RPT, C), jnp.float32)

    t1 = _tc1a(x, W1_l)
    a0, a1, a2, a3, cnt_a, cnt_b = _sc_agg1(
        t1, src_ix, dst_ix, zrow1, zcnt, ones_rows)
    r1 = _tc1b(x, W1_r, b1_row)

    t2_tbl = _tc2a((a0, a1, a2, a3), cnt_a, cnt_b, r1, W2_l)

    o0, o1 = _sc_agg2(t2_tbl, src_ix, dst_ix, zrow2)
    r2 = _tc2b((a0, a1, a2, a3), cnt_a, cnt_b, r1, W2_r, b2_row)

    return _tc3((o0, o1), cnt_a, cnt_b, r2)
